# baseline (speedup 1.0000x reference)
import jax
import jax.numpy as jnp
from jax import lax
from jax.experimental import pallas as pl
from jax.experimental.pallas import tpu as pltpu
from jax.experimental.pallas import tpu_sc as plsc

B = 16384
F = 26
V = 100000
D = 32
ND = F * D
H1 = 256
H2 = 128

NW = 32
QB = 4096
NQ = B // QB


def _gather_body(xt_hbm, tab_hbm, out_hbm, xq0, xq1, oq0, oq1, rowbuf,
                 sem_row, sem_x0, sem_x1, sem_o0, sem_o1):
    w = lax.axis_index("s") * 2 + lax.axis_index("c")
    xq = (xq0, xq1)
    sx = (sem_x0, sem_x1)
    oq = (oq0, oq1)
    so = (sem_o0, sem_o1)

    pltpu.async_copy(tab_hbm.at[w], rowbuf, sem_row)
    pltpu.async_copy(xt_hbm.at[0, pl.ds(0, QB)], xq0, sem_x0)

    def field(f, carry):
        r = f * D + w
        pltpu.make_async_copy(tab_hbm.at[r], rowbuf, sem_row).wait()
        pltpu.make_async_copy(xt_hbm.at[f, pl.ds(0, QB)], xq0, sem_x0).wait()

        odesc = {}
        for q in range(NQ):
            cur = q % 2
            if q + 1 < NQ:
                xdesc = pltpu.async_copy(
                    xt_hbm.at[f, pl.ds((q + 1) * QB, QB)], xq[(q + 1) % 2],
                    sx[(q + 1) % 2])
            if q >= 2:
                odesc[q - 2].wait()

            @plsc.parallel_loop(0, QB // 16, unroll=16)
            def g16(i):
                off = pl.multiple_of(i * 16, 16)
                idx = xq[cur][pl.ds(off, 16)]
                oq[cur][pl.ds(off, 16)] = plsc.load_gather(rowbuf, [idx])

            odesc[q] = pltpu.async_copy(
                oq[cur], out_hbm.at[r, pl.ds(q * QB, QB)], so[cur])
            if q + 1 < NQ:
                xdesc.wait()

        fn = jnp.minimum(f + 1, F - 1)
        pltpu.async_copy(tab_hbm.at[fn * D + w], rowbuf, sem_row)
        pltpu.async_copy(xt_hbm.at[fn, pl.ds(0, QB)], xq0, sem_x0)
        odesc[NQ - 2].wait()
        odesc[NQ - 1].wait()
        return carry

    lax.fori_loop(0, F, field, 0)
    pltpu.make_async_copy(tab_hbm.at[(F - 1) * D + w], rowbuf, sem_row).wait()
    pltpu.make_async_copy(xt_hbm.at[F - 1, pl.ds(0, QB)], xq0, sem_x0).wait()


_gather_cache = []


def _gather(xt, tab):
    if not _gather_cache:
        _gather_cache.append(pl.kernel(
            _gather_body,
            out_type=jax.ShapeDtypeStruct((ND, B), jnp.float32),
            mesh=plsc.VectorSubcoreMesh(core_axis_name="c", subcore_axis_name="s"),
            scratch_types=[
                pltpu.VMEM((QB,), jnp.int32),
                pltpu.VMEM((QB,), jnp.int32),
                pltpu.VMEM((QB,), jnp.float32),
                pltpu.VMEM((QB,), jnp.float32),
                pltpu.VMEM((V,), jnp.float32),
                pltpu.SemaphoreType.DMA,
                pltpu.SemaphoreType.DMA,
                pltpu.SemaphoreType.DMA,
                pltpu.SemaphoreType.DMA,
                pltpu.SemaphoreType.DMA,
            ],
            compiler_params=pltpu.CompilerParams(
                use_tc_tiling_on_sc=True, needs_layout_passes=False),
        ))
    return _gather_cache[0](xt, tab)


BK = 1024


def _mlp_body(e_ref, w1t_ref, b1_ref, w2t_ref, b2_ref, w3_ref, b3_ref, o_ref):
    e = e_ref[...]
    h = jnp.dot(w1t_ref[...], e, preferred_element_type=jnp.float32)
    h = jnp.maximum(h + b1_ref[...][:, None], 0.0)
    h = jnp.dot(w2t_ref[...], h, preferred_element_type=jnp.float32)
    h = jnp.maximum(h + b2_ref[...][:, None], 0.0)
    z = jnp.sum(h * w3_ref[...][:, None], axis=0) + b3_ref[...]
    o_ref[...] = 1.0 / (1.0 + jnp.exp(-z))


_mlp = pl.pallas_call(
    _mlp_body,
    grid=(B // BK,),
    in_specs=[
        pl.BlockSpec((ND, BK), lambda i: (0, i)),
        pl.BlockSpec((H1, ND), lambda i: (0, 0)),
        pl.BlockSpec((H1,), lambda i: (0,)),
        pl.BlockSpec((H2, H1), lambda i: (0, 0)),
        pl.BlockSpec((H2,), lambda i: (0,)),
        pl.BlockSpec((H2,), lambda i: (0,)),
        pl.BlockSpec((1,), lambda i: (0,)),
    ],
    out_specs=pl.BlockSpec((BK,), lambda i: (i,)),
    out_shape=jax.ShapeDtypeStruct((B,), jnp.float32),
    compiler_params=pltpu.CompilerParams(
        dimension_semantics=("parallel",),
    ),
)


def kernel(x, tables, W1, b1, W2, b2, W3, b3):
    xt = x.astype(jnp.int32).T
    tab = tables.transpose(0, 2, 1).reshape(ND, V)
    embT = _gather(xt, tab)
    return _mlp(embT, W1.T, b1, W2.T, b2, W3.reshape(H2), b3)


# speedup vs baseline: 1.1019x; 1.1019x over previous
import jax
import jax.numpy as jnp
from jax import lax
from jax.experimental import pallas as pl
from jax.experimental.pallas import tpu as pltpu
from jax.experimental.pallas import tpu_sc as plsc

B = 16384
F = 26
V = 100000
D = 32
ND = F * D
H1 = 256
H2 = 128

NW = 32
QB = 4096
NQ = B // QB
XH = 8192


def _gather_body(xt_hbm, tab_hbm, out_hbm, xh0, xh1, oq0, oq1, rowbuf,
                 sem_row, sem_x0, sem_x1, sem_o0, sem_o1):
    w = lax.axis_index("s") * 2 + lax.axis_index("c")
    xh = (xh0, xh1)
    sx = (sem_x0, sem_x1)
    oq = (oq0, oq1)
    so = (sem_o0, sem_o1)

    def issue_row(r):
        pltpu.async_copy(tab_hbm.at[r], rowbuf, sem_row)

    def wait_row(r):
        pltpu.make_async_copy(tab_hbm.at[r], rowbuf, sem_row).wait()

    issue_row(w)
    pltpu.async_copy(xt_hbm.at[0, pl.ds(0, XH)], xh0, sem_x0)

    def field(f, carry):
        r = f * D + w
        fn = jnp.minimum(f + 1, F - 1)
        wait_row(r)
        pltpu.make_async_copy(xt_hbm.at[f, pl.ds(0, XH)], xh0, sem_x0).wait()

        odesc = {}
        xdesc = None
        for q in range(NQ):
            if q == 0:
                xdesc = pltpu.async_copy(
                    xt_hbm.at[f, pl.ds(XH, XH)], xh1, sem_x1)
            if q == 2:
                xdesc.wait()
            if q >= 2:
                odesc[q - 2].wait()

            src = xh[q // 2]
            soff = (q % 2) * QB

            @plsc.parallel_loop(0, QB // 16, unroll=8)
            def g16(i):
                off = pl.multiple_of(soff + i * 16, 16)
                idx = src[pl.ds(off, 16)]
                oq[q % 2][pl.ds(pl.multiple_of(i * 16, 16), 16)] = (
                    plsc.load_gather(rowbuf, [idx]))

            odesc[q] = pltpu.async_copy(
                oq[q % 2], out_hbm.at[r, pl.ds(q * QB, QB)], so[q % 2])
            if q == 1:
                pltpu.async_copy(xt_hbm.at[fn, pl.ds(0, XH)], xh0, sem_x0)

        issue_row(fn * D + w)
        odesc[NQ - 2].wait()
        odesc[NQ - 1].wait()
        return carry

    lax.fori_loop(0, F, field, 0)
    wait_row((F - 1) * D + w)
    pltpu.make_async_copy(xt_hbm.at[F - 1, pl.ds(0, XH)], xh0, sem_x0).wait()


_gather_cache = []


def _gather(xt, tab):
    if not _gather_cache:
        _gather_cache.append(pl.kernel(
            _gather_body,
            out_type=jax.ShapeDtypeStruct((ND, B), jnp.float32),
            mesh=plsc.VectorSubcoreMesh(core_axis_name="c", subcore_axis_name="s"),
            scratch_types=[
                pltpu.VMEM((XH,), jnp.int32),
                pltpu.VMEM((XH,), jnp.int32),
                pltpu.VMEM((QB,), jnp.float32),
                pltpu.VMEM((QB,), jnp.float32),
                pltpu.VMEM((V,), jnp.float32),
                pltpu.SemaphoreType.DMA,
                pltpu.SemaphoreType.DMA,
                pltpu.SemaphoreType.DMA,
                pltpu.SemaphoreType.DMA,
                pltpu.SemaphoreType.DMA,
            ],
            compiler_params=pltpu.CompilerParams(
                use_tc_tiling_on_sc=True, needs_layout_passes=False),
        ))
    return _gather_cache[0](xt, tab)


BK = 1024


def _mlp_body(e_ref, w1t_ref, b1_ref, w2t_ref, b2_ref, w3_ref, b3_ref, o_ref):
    e = e_ref[...]
    h = jnp.dot(w1t_ref[...], e, preferred_element_type=jnp.float32)
    h = jnp.maximum(h + b1_ref[...][:, None], 0.0)
    h = jnp.dot(w2t_ref[...], h, preferred_element_type=jnp.float32)
    h = jnp.maximum(h + b2_ref[...][:, None], 0.0)
    z = jnp.sum(h * w3_ref[...][:, None], axis=0) + b3_ref[...]
    o_ref[...] = 1.0 / (1.0 + jnp.exp(-z))


_mlp = pl.pallas_call(
    _mlp_body,
    grid=(B // BK,),
    in_specs=[
        pl.BlockSpec((ND, BK), lambda i: (0, i)),
        pl.BlockSpec((H1, ND), lambda i: (0, 0)),
        pl.BlockSpec((H1,), lambda i: (0,)),
        pl.BlockSpec((H2, H1), lambda i: (0, 0)),
        pl.BlockSpec((H2,), lambda i: (0,)),
        pl.BlockSpec((H2,), lambda i: (0,)),
        pl.BlockSpec((1,), lambda i: (0,)),
    ],
    out_specs=pl.BlockSpec((BK,), lambda i: (i,)),
    out_shape=jax.ShapeDtypeStruct((B,), jnp.float32),
    compiler_params=pltpu.CompilerParams(
        dimension_semantics=("parallel",),
    ),
)


def kernel(x, tables, W1, b1, W2, b2, W3, b3):
    xt = x.astype(jnp.int32).T
    tab = tables.transpose(0, 2, 1).reshape(ND, V)
    embT = _gather(xt, tab)
    return _mlp(embT, W1.T, b1, W2.T, b2, W3.reshape(H2), b3)


# speedup vs baseline: 1.1172x; 1.0138x over previous
import jax
import jax.numpy as jnp
from jax import lax
from jax.experimental import pallas as pl
from jax.experimental.pallas import tpu as pltpu
from jax.experimental.pallas import tpu_sc as plsc

B = 16384
F = 26
V = 100000
D = 32
ND = F * D
H1 = 256
H2 = 128

NW = 32
QB = 4096
NQ = B // QB
XH = 8192


def _gather_body(xt_hbm, tab_hbm, out_hbm, xh0, xh1, oq0, oq1, rowbuf,
                 sem_row, sem_x0, sem_x1, sem_o0, sem_o1):
    w = lax.axis_index("s") * 2 + lax.axis_index("c")
    xh = (xh0, xh1)
    sx = (sem_x0, sem_x1)
    oq = (oq0, oq1)
    so = (sem_o0, sem_o1)

    def issue_row(r):
        pltpu.async_copy(tab_hbm.at[r], rowbuf, sem_row)

    def wait_row(r):
        pltpu.make_async_copy(tab_hbm.at[r], rowbuf, sem_row).wait()

    issue_row(w)
    pltpu.async_copy(xt_hbm.at[0, pl.ds(0, XH)], xh0, sem_x0)

    def field(f, carry):
        r = f * D + w
        fn = jnp.minimum(f + 1, F - 1)
        wait_row(r)
        pltpu.make_async_copy(xt_hbm.at[f, pl.ds(0, XH)], xh0, sem_x0).wait()

        odesc = {}
        xdesc = None
        for q in range(NQ):
            if q == 0:
                xdesc = pltpu.async_copy(
                    xt_hbm.at[f, pl.ds(XH, XH)], xh1, sem_x1)
            if q == 2:
                xdesc.wait()
            if q >= 2:
                odesc[q - 2].wait()

            src = xh[q // 2]
            soff = (q % 2) * QB

            @plsc.parallel_loop(0, QB // 16, unroll=8)
            def g16(i):
                off = pl.multiple_of(soff + i * 16, 16)
                idx = src[pl.ds(off, 16)]
                oq[q % 2][pl.ds(pl.multiple_of(i * 16, 16), 16)] = (
                    plsc.load_gather(rowbuf, [idx]))

            odesc[q] = pltpu.async_copy(
                oq[q % 2], out_hbm.at[r, pl.ds(q * QB, QB)], so[q % 2])
            if q == 1:
                pltpu.async_copy(xt_hbm.at[fn, pl.ds(0, XH)], xh0, sem_x0)

        issue_row(fn * D + w)
        odesc[NQ - 2].wait()
        odesc[NQ - 1].wait()
        return carry

    lax.fori_loop(0, F, field, 0)
    wait_row((F - 1) * D + w)
    pltpu.make_async_copy(xt_hbm.at[F - 1, pl.ds(0, XH)], xh0, sem_x0).wait()


_gather_cache = []


def _gather(xt, tab):
    if not _gather_cache:
        _gather_cache.append(pl.kernel(
            _gather_body,
            out_type=jax.ShapeDtypeStruct((ND, B), jnp.float32),
            mesh=plsc.VectorSubcoreMesh(core_axis_name="c", subcore_axis_name="s"),
            scratch_types=[
                pltpu.VMEM((XH,), jnp.int32),
                pltpu.VMEM((XH,), jnp.int32),
                pltpu.VMEM((QB,), jnp.float32),
                pltpu.VMEM((QB,), jnp.float32),
                pltpu.VMEM((V,), jnp.float32),
                pltpu.SemaphoreType.DMA,
                pltpu.SemaphoreType.DMA,
                pltpu.SemaphoreType.DMA,
                pltpu.SemaphoreType.DMA,
                pltpu.SemaphoreType.DMA,
            ],
            compiler_params=pltpu.CompilerParams(
                use_tc_tiling_on_sc=True, needs_layout_passes=False),
        ))
    return _gather_cache[0](xt, tab)


BK = 2048


def _mlp_body(e_ref, w1t_ref, b1_ref, w2t_ref, b2_ref, w3_ref, b3_ref, o_ref):
    e = e_ref[...]
    h = jnp.dot(w1t_ref[...], e, preferred_element_type=jnp.float32)
    h = jnp.maximum(h + b1_ref[...][:, None], 0.0)
    h = jnp.dot(w2t_ref[...], h, preferred_element_type=jnp.float32)
    h = jnp.maximum(h + b2_ref[...][:, None], 0.0)
    z = jnp.sum(h * w3_ref[...][:, None], axis=0) + b3_ref[...]
    o_ref[...] = 1.0 / (1.0 + jnp.exp(-z))


_mlp = pl.pallas_call(
    _mlp_body,
    grid=(B // BK,),
    in_specs=[
        pl.BlockSpec((ND, BK), lambda i: (0, i)),
        pl.BlockSpec((H1, ND), lambda i: (0, 0)),
        pl.BlockSpec((H1,), lambda i: (0,)),
        pl.BlockSpec((H2, H1), lambda i: (0, 0)),
        pl.BlockSpec((H2,), lambda i: (0,)),
        pl.BlockSpec((H2,), lambda i: (0,)),
        pl.BlockSpec((1,), lambda i: (0,)),
    ],
    out_specs=pl.BlockSpec((BK,), lambda i: (i,)),
    out_shape=jax.ShapeDtypeStruct((B,), jnp.float32),
    compiler_params=pltpu.CompilerParams(
        dimension_semantics=("parallel",),
    ),
)


def kernel(x, tables, W1, b1, W2, b2, W3, b3):
    xt = x.astype(jnp.int32).T
    tab = tables.transpose(0, 2, 1).reshape(ND, V)
    embT = _gather(xt, tab)
    return _mlp(embT, W1.T, b1, W2.T, b2, W3.reshape(H2), b3)
